# Initial kernel scaffold; baseline (speedup 1.0000x reference)
import jax
import jax.numpy as jnp
from jax.experimental import pallas as pl


def kernel(xyz, params):
    raise NotImplementedError("write your pallas kernel here")


# speedup vs baseline: 3.1813x; 3.1813x over previous
import functools

import jax
import jax.numpy as jnp
from jax.experimental import pallas as pl

_F32 = jnp.float32
_HI = jax.lax.Precision.HIGHEST


def _flatten_ws(layers):
    ws = []
    for l in layers:
        ws += [l['W'], l['b'].reshape(1, -1), l['mean'].reshape(1, -1),
               l['var'].reshape(1, -1), l['gamma'].reshape(1, -1),
               l['beta'].reshape(1, -1)]
    return ws


def _mlp_chain(x, ws):
    for i in range(0, len(ws), 6):
        W, b, mean, var, gamma, beta = ws[i:i + 6]
        y = jnp.dot(x, W[...], preferred_element_type=_F32) + b[...]
        y = (y - mean[...]) / jnp.sqrt(var[...] + 1e-5) * gamma[...] + beta[...]
        x = jnp.maximum(y, 0.0)
    return x


def _fps_body(npoint, B, N, coords_ref, out_ref):
    X = coords_ref[0:B, :]
    Y = coords_ref[B:2 * B, :]
    Z = coords_ref[2 * B:3 * B, :]
    lane_n = jax.lax.broadcasted_iota(jnp.int32, (B, N), 1)
    lane_s = jax.lax.broadcasted_iota(jnp.int32, (B, npoint), 1)

    def body(i, st):
        dist, far, ax, ay, az = st
        oh = (lane_n == far).astype(_F32)
        cx = jnp.sum(X * oh, axis=1, keepdims=True)
        cy = jnp.sum(Y * oh, axis=1, keepdims=True)
        cz = jnp.sum(Z * oh, axis=1, keepdims=True)
        sel = lane_s == i
        ax = jnp.where(sel, cx, ax)
        ay = jnp.where(sel, cy, ay)
        az = jnp.where(sel, cz, az)
        dx = X - cx
        dy = Y - cy
        dz = Z - cz
        d = (dx * dx + dy * dy) + dz * dz
        dist = jnp.minimum(dist, d)
        mx = jnp.max(dist, axis=1, keepdims=True)
        far = jnp.min(jnp.where(dist == mx, lane_n, N), axis=1, keepdims=True)
        return dist, far, ax, ay, az

    z = jnp.zeros((B, npoint), _F32)
    st = (jnp.full((B, N), 1e10, _F32), jnp.zeros((B, 1), jnp.int32), z, z, z)
    _, _, ax, ay, az = jax.lax.fori_loop(0, npoint, body, st)
    out_ref[0:B, :] = ax
    out_ref[B:2 * B, :] = ay
    out_ref[2 * B:3 * B, :] = az


def _fps(coords, npoint):
    threeb, n = coords.shape
    b = threeb // 3
    return pl.pallas_call(
        functools.partial(_fps_body, npoint, b, n),
        out_shape=jax.ShapeDtypeStruct((threeb, npoint), _F32),
    )(coords)


def _sa_body(r2, nsample, Nc, nw, centN_ref, coordsT_ref, g_ref, *rest):
    ws = rest[:nw]
    out_ref = rest[nw]
    cn = centN_ref[0]
    xt = coordsT_ref[0]
    G = g_ref[0]
    Sblk = cn.shape[0]
    N = xt.shape[1]
    Cg = G.shape[1]
    sq_c = jnp.sum(cn * cn, axis=1, keepdims=True)
    sq_x = jnp.sum(xt * xt, axis=0)[None, :]
    ca = cn.astype(jnp.bfloat16).astype(_F32)
    xa = xt.astype(jnp.bfloat16).astype(_F32)
    dots = (ca[:, 0:1] * xa[0][None, :]
            + ca[:, 1:2] * xa[1][None, :]
            + ca[:, 2:3] * xa[2][None, :])
    d2 = (sq_c + sq_x) - 2.0 * dots
    msk = d2 <= r2
    c = msk.astype(_F32)
    k = 1
    while k < N:
        c = c + jnp.concatenate(
            [jnp.zeros((Sblk, k), _F32), c[:, :N - k]], axis=1)
        k *= 2
    count = c[:, N - 1:N]
    jio = jax.lax.broadcasted_iota(jnp.int32, (1, nsample, 1), 1).astype(_F32)
    acc = jnp.zeros((Sblk * nsample, Cg), _F32)
    for n0 in range(0, N, Nc):
        rank = c[:, n0:n0 + Nc][:, None, :]
        mm = msk[:, n0:n0 + Nc][:, None, :]
        sel = jnp.logical_and(rank == (jio + 1.0), mm)
        P = sel.astype(_F32).reshape(Sblk * nsample, Nc)
        acc = acc + jnp.dot(P, G[n0:n0 + Nc, :], preferred_element_type=_F32, precision=_HI)
    gath = acc.reshape(Sblk, nsample, Cg)
    gx = gath[:, :, 0:3] - cn[:, None, :]
    x = jnp.concatenate([gx, gath[:, :, 3:]], axis=2)
    x = x.reshape(Sblk * nsample, Cg)
    h = _mlp_chain(x, ws)
    c3 = h.shape[1]
    h3 = h.reshape(Sblk, nsample, c3)
    valid = jio < count[:, :, None]
    out_ref[0] = jnp.max(jnp.where(valid, h3, -jnp.inf), axis=1)


def _sa(centN, coordsT, G, layers, r2, nsample, Sblk, Nc):
    B, S, _ = centN.shape
    N = coordsT.shape[2]
    Cg = G.shape[2]
    ws = _flatten_ws(layers)
    nw = len(ws)
    c3 = layers[-1]['W'].shape[1]
    in_specs = [
        pl.BlockSpec((1, Sblk, 3), lambda b, s: (b, s, 0)),
        pl.BlockSpec((1, 3, N), lambda b, s: (b, 0, 0)),
        pl.BlockSpec((1, N, Cg), lambda b, s: (b, 0, 0)),
    ] + [pl.BlockSpec(w.shape, lambda b, s: (0, 0)) for w in ws]
    return pl.pallas_call(
        functools.partial(_sa_body, r2, nsample, Nc, nw),
        grid=(B, S // Sblk),
        in_specs=in_specs,
        out_specs=pl.BlockSpec((1, Sblk, c3), lambda b, s: (b, s, 0)),
        out_shape=jax.ShapeDtypeStruct((B, S, c3), _F32),
    )(centN, coordsT, G, *ws)


def _sa_all_body(nw, xyz_ref, p_ref, *rest):
    ws = rest[:nw]
    out_ref = rest[nw]
    x = jnp.concatenate([xyz_ref[0], p_ref[0]], axis=1)
    h = _mlp_chain(x, ws)
    out_ref[0] = jnp.max(h, axis=0, keepdims=True)


def _sa_all(xyzN, p, layers):
    B, S, _ = xyzN.shape
    ws = _flatten_ws(layers)
    nw = len(ws)
    c3 = layers[-1]['W'].shape[1]
    in_specs = [
        pl.BlockSpec((1, S, 3), lambda b: (b, 0, 0)),
        pl.BlockSpec((1, S, p.shape[2]), lambda b: (b, 0, 0)),
    ] + [pl.BlockSpec(w.shape, lambda b: (0, 0)) for w in ws]
    return pl.pallas_call(
        functools.partial(_sa_all_body, nw),
        grid=(B,),
        in_specs=in_specs,
        out_specs=pl.BlockSpec((1, 1, c3), lambda b: (b, 0, 0)),
        out_shape=jax.ShapeDtypeStruct((B, 1, c3), _F32),
    )(xyzN, p, *ws)


def _fp_bcast_body(nw, p1_ref, p2_ref, *rest):
    ws = rest[:nw]
    out_ref = rest[nw]
    p1 = p1_ref[0]
    p2 = p2_ref[0]
    x = jnp.concatenate(
        [p1, jnp.broadcast_to(p2, (p1.shape[0], p2.shape[1]))], axis=1)
    out_ref[0] = _mlp_chain(x, ws)


def _fp_bcast(p1, p2, layers):
    B, S, C1 = p1.shape
    ws = _flatten_ws(layers)
    nw = len(ws)
    cout = layers[-1]['W'].shape[1]
    in_specs = [
        pl.BlockSpec((1, S, C1), lambda b: (b, 0, 0)),
        pl.BlockSpec((1, 1, p2.shape[2]), lambda b: (b, 0, 0)),
    ] + [pl.BlockSpec(w.shape, lambda b: (0, 0)) for w in ws]
    return pl.pallas_call(
        functools.partial(_fp_bcast_body, nw),
        grid=(B,),
        in_specs=in_specs,
        out_specs=pl.BlockSpec((1, S, cout), lambda b: (b, 0, 0)),
        out_shape=jax.ShapeDtypeStruct((B, S, cout), _F32),
    )(p1, p2, *ws)


def _fp_nn_body(nw, x1T_ref, x2T_ref, p1_ref, p2_ref, *rest):
    ws = rest[:nw]
    out_ref = rest[nw]
    x1 = x1T_ref[0]
    x2 = x2T_ref[0]
    p1 = p1_ref[0]
    p2 = p2_ref[0]
    R = x1.shape[1]
    S2 = x2.shape[1]
    sq1 = jnp.sum(x1 * x1, axis=0)[:, None]
    sq2 = jnp.sum(x2 * x2, axis=0)[None, :]
    a = x1.astype(jnp.bfloat16).astype(_F32)
    b = x2.astype(jnp.bfloat16).astype(_F32)
    dots = (a[0][:, None] * b[0][None, :]
            + a[1][:, None] * b[1][None, :]
            + a[2][:, None] * b[2][None, :])
    d = (sq1 + sq2) - 2.0 * dots
    lane = jax.lax.broadcasted_iota(jnp.int32, (R, S2), 1)
    dcur = d
    sels = []
    for _ in range(3):
        dk = jnp.min(dcur, axis=1, keepdims=True)
        ik = jnp.min(jnp.where(dcur == dk, lane, S2), axis=1, keepdims=True)
        oh = lane == ik
        sels.append((oh, 1.0 / (dk + 1e-8)))
        dcur = jnp.where(oh, jnp.inf, dcur)
    norm = sels[0][1] + sels[1][1] + sels[2][1]
    interp = None
    for oh, rk in sels:
        row = jnp.dot(oh.astype(_F32), p2, preferred_element_type=_F32,
                      precision=_HI)
        term = (rk / norm) * row
        interp = term if interp is None else interp + term
    x = jnp.concatenate([p1, interp], axis=1)
    out_ref[0] = _mlp_chain(x, ws)


def _fp_nn(x1T, x2T, p1, p2, layers, Rblk):
    B, _, R = x1T.shape
    S2 = x2T.shape[2]
    C1 = p1.shape[2]
    C2 = p2.shape[2]
    ws = _flatten_ws(layers)
    nw = len(ws)
    cout = layers[-1]['W'].shape[1]
    in_specs = [
        pl.BlockSpec((1, 3, Rblk), lambda b, r: (b, 0, r)),
        pl.BlockSpec((1, 3, S2), lambda b, r: (b, 0, 0)),
        pl.BlockSpec((1, Rblk, C1), lambda b, r: (b, r, 0)),
        pl.BlockSpec((1, S2, C2), lambda b, r: (b, 0, 0)),
    ] + [pl.BlockSpec(w.shape, lambda b, r: (0, 0)) for w in ws]
    return pl.pallas_call(
        functools.partial(_fp_nn_body, nw),
        grid=(B, R // Rblk),
        in_specs=in_specs,
        out_specs=pl.BlockSpec((1, Rblk, cout), lambda b, r: (b, r, 0)),
        out_shape=jax.ShapeDtypeStruct((B, R, cout), _F32),
    )(x1T, x2T, p1, p2, *ws)


def _head_body(h0_ref, h1_ref, *rest):
    wr = rest[:-1]
    out_ref = rest[-1]
    f0 = jnp.dot(h0_ref[0], wr[0][...], preferred_element_type=_F32) + wr[1][...]
    f1 = jnp.dot(h1_ref[0], wr[2][...], preferred_element_type=_F32) + wr[3][...]
    feat = jnp.concatenate([f0, f1], axis=1)
    for i in range(4, len(wr), 2):
        feat = jnp.dot(feat, wr[i][...],
                       preferred_element_type=_F32) + wr[i + 1][...]
    out_ref[0] = feat


def _head(h0, h1, convs):
    B, N, C = h0.shape
    ws = []
    for cv in convs:
        ws += [cv['W'], cv['b'].reshape(1, -1)]
    in_specs = [
        pl.BlockSpec((1, N, C), lambda b: (b, 0, 0)),
        pl.BlockSpec((1, N, C), lambda b: (b, 0, 0)),
    ] + [pl.BlockSpec(w.shape, lambda b: (0, 0)) for w in ws]
    return pl.pallas_call(
        _head_body,
        grid=(B,),
        in_specs=in_specs,
        out_specs=pl.BlockSpec((1, N, 1), lambda b: (b, 0, 0)),
        out_shape=jax.ShapeDtypeStruct((B, N, 1), _F32),
    )(h0, h1, *ws)


def kernel(xyz, params):
    B, _, N = xyz.shape

    sa1w, sa2w, sa3w = params['sa1'], params['sa2'], params['sa3']
    sa4w, sa5w, sa6w = params['sa4'], params['sa5'], params['sa6']
    fp1w, fp2w, fp3w = params['fp1'], params['fp2'], params['fp3']
    fp4w, fp5w, fp6w = params['fp4'], params['fp5'], params['fp6']
    convs = [params[nm] for nm in
             ('conv0', 'conv1', 'conv2', 'conv3', 'conv4', 'conv5', 'conv6')]

    l0N = xyz.transpose(0, 2, 1)
    coords0 = xyz.transpose(1, 0, 2).reshape(3 * B, N)
    l1c = _fps(coords0, 512)
    l1T = l1c.reshape(3, B, 512).transpose(1, 0, 2)
    l1N = l1c.reshape(3, B, 512).transpose(1, 2, 0)
    l2c = _fps(l1c, 128)
    l2T = l2c.reshape(3, B, 128).transpose(1, 0, 2)
    l2N = l2c.reshape(3, B, 128).transpose(1, 2, 0)

    G1 = jnp.concatenate([l0N, l0N], axis=2)

    l1p0 = _sa(l1N, xyz, G1, sa1w, 0.2 ** 2, 64, 8, 512)
    G2_0 = jnp.concatenate([l1N, l1p0], axis=2)
    l2p0 = _sa(l2N, l1T, G2_0, sa2w, 0.4 ** 2, 64, 16, 512)
    l3p0 = _sa_all(l2N, l2p0, sa3w)
    l2p0 = _fp_bcast(l2p0, l3p0, fp3w)
    l1p0 = _fp_nn(l1T, l2T, l1p0, l2p0, fp2w, 512)
    h0 = _fp_nn(xyz, l1T, G1, l1p0, fp1w, 512)

    l1p1 = _sa(l1N, xyz, G1, sa4w, 0.3 ** 2, 64, 8, 512)
    G2_1 = jnp.concatenate([l1N, l1p1], axis=2)
    l2p1 = _sa(l2N, l1T, G2_1, sa5w, 0.5 ** 2, 64, 16, 512)
    l3p1 = _sa_all(l2N, l2p1, sa6w)
    l2p1 = _fp_bcast(l2p1, l3p1, fp4w)
    l1p1 = _fp_nn(l1T, l2T, l1p1, l2p1, fp5w, 512)
    h1 = _fp_nn(xyz, l1T, G1, l1p1, fp6w, 512)

    return _head(h0, h1, convs)
